# baseline (speedup 1.0000x reference)
import jax
import jax.numpy as jnp
from jax import lax
from jax.experimental import pallas as pl
from jax.experimental.pallas import tpu as pltpu
from jax.experimental.pallas import tpu_sc as plsc

EMBED = 512
NC, NS = 2, 16
NW = NC * NS
NQ = 4
CB = 32


def _body(table_hbm, idx_hbm, out_hbm, idx_v, rows_v, *sems):
    hist, bpw = idx_v.shape
    wid = lax.axis_index("s") * NC + lax.axis_index("c")
    b0 = wid * bpw
    pltpu.sync_copy(idx_hbm.at[:, pl.ds(b0, bpw)], idx_v)

    gsems, ssems = sems[:NQ], sems[NQ:]

    def gstart(h, q):
        pltpu.async_copy(
            table_hbm.at[idx_v.at[h].at[pl.ds(q * CB, CB)]],
            rows_v.at[q], gsems[q])

    def gwait(h, q):
        pltpu.make_async_copy(
            table_hbm.at[idx_v.at[h].at[pl.ds(q * CB, CB)]],
            rows_v.at[q], gsems[q]).wait()

    def sstart(h, q):
        pltpu.async_copy(
            rows_v.at[q], out_hbm.at[h].at[pl.ds(b0 + q * CB, CB)],
            ssems[q])

    def swait(h, q):
        pltpu.make_async_copy(
            rows_v.at[q], out_hbm.at[h].at[pl.ds(b0 + q * CB, CB)],
            ssems[q]).wait()

    def step(h, q, first=False, last=False):
        if not first:
            ph, pq = (h, q - 1) if q > 0 else (h - 1, NQ - 1)
            swait(ph, pq)
        if not last:
            nh, nq = (h, NQ - 1) if q == 0 else (h + 1, q - 1)
            gstart(nh, nq)
        gwait(h, q)
        sstart(h, q)

    for q in range(NQ - 1):
        gstart(0, q)
    for q in range(NQ):
        step(0, q, first=(q == 0))

    @pl.loop(1, hist - 1)
    def _(h):
        for q in range(NQ):
            step(h, q)

    for q in range(NQ):
        step(hist - 1, q, last=(q > 0))
    swait(hist - 1, NQ - 1)


def kernel(x, embedding, feedforward_0, feedforward_1, feedforward_2,
           feedforward_3):
    batch, hist = x.shape
    xt = jnp.swapaxes(x.astype(jnp.int32), 0, 1)
    bpw = batch // NW

    mesh = plsc.VectorSubcoreMesh(
        core_axis_name="c", subcore_axis_name="s",
        num_cores=NC, num_subcores=NS)
    gather = pl.kernel(
        _body,
        out_type=jax.ShapeDtypeStruct((hist, batch, EMBED), jnp.float32),
        mesh=mesh,
        scratch_types=[
            pltpu.VMEM((hist, bpw), jnp.int32),
            pltpu.VMEM((NQ, CB, EMBED), jnp.float32),
        ] + [pltpu.SemaphoreType.DMA] * (2 * NQ),
    )
    out = gather(embedding, xt)
    return jnp.transpose(out, (1, 0, 2))


# speedup vs baseline: 1.0055x; 1.0055x over previous
import jax
import jax.numpy as jnp
from jax import lax
from jax.experimental import pallas as pl
from jax.experimental.pallas import tpu as pltpu
from jax.experimental.pallas import tpu_sc as plsc

EMBED = 512
NC, NS = 2, 16
NW = NC * NS
CB = 64


def _body(table_hbm, idx_hbm, out_hbm, idx_v, rows_v,
          gsem0, gsem1, ssem0, ssem1):
    hist, bpw = idx_v.shape
    wid = lax.axis_index("s") * NC + lax.axis_index("c")
    b0 = wid * bpw
    pltpu.sync_copy(idx_hbm.at[wid], idx_v)

    gsems = (gsem0, gsem1)
    ssems = (ssem0, ssem1)

    def gstart(h, half, b):
        pltpu.async_copy(
            table_hbm.at[idx_v.at[h].at[pl.ds(half * CB, CB)]],
            rows_v.at[b], gsems[b])

    def gwait(h, half, b):
        pltpu.make_async_copy(
            table_hbm.at[idx_v.at[h].at[pl.ds(half * CB, CB)]],
            rows_v.at[b], gsems[b]).wait()

    def sstart(h, half, b):
        pltpu.async_copy(
            rows_v.at[b], out_hbm.at[h].at[pl.ds(b0 + half * CB, CB)],
            ssems[b])

    def swait(h, half, b):
        pltpu.make_async_copy(
            rows_v.at[b], out_hbm.at[h].at[pl.ds(b0 + half * CB, CB)],
            ssems[b]).wait()

    nhalf = bpw // CB

    gstart(0, 0, 0)
    gstart(0, 1, 1)
    gwait(0, 0, 0)
    sstart(0, 0, 0)

    @pl.loop(1, hist)
    def _(h):
        for half in range(nhalf):
            b = half
            swait(h - 1, half, b)
            gstart(h, half, b)
            ph, phalf = (h, 0) if half == 1 else (h - 1, 1)
            gwait(ph, phalf, 1 - b)
            sstart(ph, phalf, 1 - b)

    gwait(hist - 1, 1, 1)
    sstart(hist - 1, 1, 1)
    swait(hist - 1, 0, 0)
    swait(hist - 1, 1, 1)


def kernel(x, embedding, feedforward_0, feedforward_1, feedforward_2,
           feedforward_3):
    batch, hist = x.shape
    bpw = batch // NW
    xw = jnp.swapaxes(x.astype(jnp.int32), 0, 1)
    xw = xw.reshape(hist, NW, bpw).transpose(1, 0, 2)

    mesh = plsc.VectorSubcoreMesh(
        core_axis_name="c", subcore_axis_name="s",
        num_cores=NC, num_subcores=NS)
    gather = pl.kernel(
        _body,
        out_type=jax.ShapeDtypeStruct((hist, batch, EMBED), jnp.float32),
        mesh=mesh,
        scratch_types=[
            pltpu.VMEM((hist, bpw), jnp.int32),
            pltpu.VMEM((2, CB, EMBED), jnp.float32),
            pltpu.SemaphoreType.DMA,
            pltpu.SemaphoreType.DMA,
            pltpu.SemaphoreType.DMA,
            pltpu.SemaphoreType.DMA,
        ],
    )
    out = gather(embedding, xw)
    return jnp.transpose(out, (1, 0, 2))


# speedup vs baseline: 1.0088x; 1.0032x over previous
import jax
import jax.numpy as jnp
from jax import lax
from jax.experimental import pallas as pl
from jax.experimental.pallas import tpu as pltpu
from jax.experimental.pallas import tpu_sc as plsc

EMBED = 512
NC, NS = 2, 16
NW = NC * NS
CB = 64


def _body(hist, bpw, table_hbm, idx_hbm, out_hbm, idx_v, rows_v,
          gsem0, gsem1, ssem0, ssem1):
    wid = lax.axis_index("s") * NC + lax.axis_index("c")
    b0 = wid * bpw
    pltpu.sync_copy(idx_hbm.at[pl.ds(wid * hist * bpw, hist * bpw)], idx_v)

    gsems = (gsem0, gsem1)
    ssems = (ssem0, ssem1)

    def gstart(h, half, b):
        pltpu.async_copy(
            table_hbm.at[idx_v.at[pl.ds(h * bpw + half * CB, CB)]],
            rows_v.at[b], gsems[b])

    def gwait(h, half, b):
        pltpu.make_async_copy(
            table_hbm.at[idx_v.at[pl.ds(h * bpw + half * CB, CB)]],
            rows_v.at[b], gsems[b]).wait()

    def sstart(h, half, b):
        pltpu.async_copy(
            rows_v.at[b], out_hbm.at[h].at[pl.ds(b0 + half * CB, CB)],
            ssems[b])

    def swait(h, half, b):
        pltpu.make_async_copy(
            rows_v.at[b], out_hbm.at[h].at[pl.ds(b0 + half * CB, CB)],
            ssems[b]).wait()

    nhalf = bpw // CB

    gstart(0, 0, 0)
    gstart(0, 1, 1)
    gwait(0, 0, 0)
    sstart(0, 0, 0)

    @pl.loop(1, hist)
    def _(h):
        for half in range(nhalf):
            b = half
            swait(h - 1, half, b)
            gstart(h, half, b)
            ph, phalf = (h, 0) if half == 1 else (h - 1, 1)
            gwait(ph, phalf, 1 - b)
            sstart(ph, phalf, 1 - b)

    gwait(hist - 1, 1, 1)
    sstart(hist - 1, 1, 1)
    swait(hist - 1, 0, 0)
    swait(hist - 1, 1, 1)


def kernel(x, embedding, feedforward_0, feedforward_1, feedforward_2,
           feedforward_3):
    batch, hist = x.shape
    bpw = batch // NW
    xw = jnp.swapaxes(x.astype(jnp.int32), 0, 1)
    xw = xw.reshape(hist, NW, bpw).transpose(1, 0, 2).reshape(-1)

    mesh = plsc.VectorSubcoreMesh(
        core_axis_name="c", subcore_axis_name="s",
        num_cores=NC, num_subcores=NS)
    import functools
    gather = pl.kernel(
        functools.partial(_body, hist, bpw),
        out_type=jax.ShapeDtypeStruct((hist, batch, EMBED), jnp.float32),
        mesh=mesh,
        scratch_types=[
            pltpu.VMEM((hist * bpw,), jnp.int32),
            pltpu.VMEM((2, CB, EMBED), jnp.float32),
            pltpu.SemaphoreType.DMA,
            pltpu.SemaphoreType.DMA,
            pltpu.SemaphoreType.DMA,
            pltpu.SemaphoreType.DMA,
        ],
    )
    out = gather(embedding, xw)
    return jnp.transpose(out, (1, 0, 2))


# speedup vs baseline: 1.5806x; 1.5668x over previous
import jax
import jax.numpy as jnp
from jax import lax
from jax.experimental import pallas as pl
from jax.experimental.pallas import tpu as pltpu
from jax.experimental.pallas import tpu_sc as plsc

EMBED = 512
KREP = 8
NC, NS = 2, 16
NW = NC * NS
CB = 64


def _body(hist, bpw, table_hbm, idx_hbm, out_hbm, idx_v, rows_v,
          gsem0, gsem1, ssem0, ssem1):
    wid = lax.axis_index("s") * NC + lax.axis_index("c")
    b0 = wid * bpw
    pltpu.sync_copy(idx_hbm.at[pl.ds(wid * hist * bpw, hist * bpw)], idx_v)

    gsems = (gsem0, gsem1)
    ssems = (ssem0, ssem1)

    def gstart(h, half, b):
        pltpu.async_copy(
            table_hbm.at[idx_v.at[pl.ds(h * bpw + half * CB, CB)]],
            rows_v.at[b], gsems[b])

    def gwait(h, half, b):
        pltpu.make_async_copy(
            table_hbm.at[idx_v.at[pl.ds(h * bpw + half * CB, CB)]],
            rows_v.at[b], gsems[b]).wait()

    def sstart(h, half, b):
        pltpu.async_copy(
            rows_v.at[b], out_hbm.at[h].at[pl.ds(b0 + half * CB, CB)],
            ssems[b])

    def swait(h, half, b):
        pltpu.make_async_copy(
            rows_v.at[b], out_hbm.at[h].at[pl.ds(b0 + half * CB, CB)],
            ssems[b]).wait()

    nhalf = bpw // CB

    gstart(0, 0, 0)
    gstart(0, 1, 1)
    gwait(0, 0, 0)
    sstart(0, 0, 0)

    @pl.loop(1, hist)
    def _(h):
        for half in range(nhalf):
            b = half
            swait(h - 1, half, b)
            gstart(h, half, b)
            ph, phalf = (h, 0) if half == 1 else (h - 1, 1)
            gwait(ph, phalf, 1 - b)
            sstart(ph, phalf, 1 - b)

    gwait(hist - 1, 1, 1)
    sstart(hist - 1, 1, 1)
    swait(hist - 1, 0, 0)
    swait(hist - 1, 1, 1)


def kernel(x, embedding, feedforward_0, feedforward_1, feedforward_2,
           feedforward_3):
    batch, hist = x.shape
    bpw = batch // NW
    xw = jnp.swapaxes(x.astype(jnp.int32), 0, 1)
    xw = xw.reshape(hist, NW, bpw).transpose(1, 0, 2)
    vocab = embedding.shape[0]
    w = jnp.arange(NW)[:, None, None]
    h = jnp.arange(hist)[None, :, None]
    b = jnp.arange(bpw)[None, None, :]
    chunk = h * (bpw // CB) + b // CB
    xw = (xw + ((w + chunk) % KREP) * vocab).reshape(-1)
    table_rep = jnp.tile(embedding, (KREP, 1))

    mesh = plsc.VectorSubcoreMesh(
        core_axis_name="c", subcore_axis_name="s",
        num_cores=NC, num_subcores=NS)
    import functools
    gather = pl.kernel(
        functools.partial(_body, hist, bpw),
        out_type=jax.ShapeDtypeStruct((hist, batch, EMBED), jnp.float32),
        mesh=mesh,
        scratch_types=[
            pltpu.VMEM((hist * bpw,), jnp.int32),
            pltpu.VMEM((2, CB, EMBED), jnp.float32),
            pltpu.SemaphoreType.DMA,
            pltpu.SemaphoreType.DMA,
            pltpu.SemaphoreType.DMA,
            pltpu.SemaphoreType.DMA,
        ],
    )
    out = gather(table_rep, xw)
    return jnp.transpose(out, (1, 0, 2))


# speedup vs baseline: 1.6296x; 1.0310x over previous
import jax
import jax.numpy as jnp
from jax import lax
from jax.experimental import pallas as pl
from jax.experimental.pallas import tpu as pltpu
from jax.experimental.pallas import tpu_sc as plsc

EMBED = 512
KREP = 16
NC, NS = 2, 16
NW = NC * NS
CB = 64


def _body(hist, bpw, table_hbm, idx_hbm, out_hbm, idx_v, rows_v,
          gsem0, gsem1, ssem0, ssem1):
    wid = lax.axis_index("s") * NC + lax.axis_index("c")
    b0 = wid * bpw
    pltpu.sync_copy(idx_hbm.at[pl.ds(wid * hist * bpw, hist * bpw)], idx_v)

    gsems = (gsem0, gsem1)
    ssems = (ssem0, ssem1)

    def gstart(h, half, b):
        pltpu.async_copy(
            table_hbm.at[idx_v.at[pl.ds(h * bpw + half * CB, CB)]],
            rows_v.at[b], gsems[b])

    def gwait(h, half, b):
        pltpu.make_async_copy(
            table_hbm.at[idx_v.at[pl.ds(h * bpw + half * CB, CB)]],
            rows_v.at[b], gsems[b]).wait()

    def sstart(h, half, b):
        pltpu.async_copy(
            rows_v.at[b], out_hbm.at[h].at[pl.ds(b0 + half * CB, CB)],
            ssems[b])

    def swait(h, half, b):
        pltpu.make_async_copy(
            rows_v.at[b], out_hbm.at[h].at[pl.ds(b0 + half * CB, CB)],
            ssems[b]).wait()

    nhalf = bpw // CB

    gstart(0, 0, 0)
    gstart(0, 1, 1)
    gwait(0, 0, 0)
    sstart(0, 0, 0)

    @pl.loop(1, hist)
    def _(h):
        for half in range(nhalf):
            b = half
            swait(h - 1, half, b)
            gstart(h, half, b)
            ph, phalf = (h, 0) if half == 1 else (h - 1, 1)
            gwait(ph, phalf, 1 - b)
            sstart(ph, phalf, 1 - b)

    gwait(hist - 1, 1, 1)
    sstart(hist - 1, 1, 1)
    swait(hist - 1, 0, 0)
    swait(hist - 1, 1, 1)


def kernel(x, embedding, feedforward_0, feedforward_1, feedforward_2,
           feedforward_3):
    batch, hist = x.shape
    bpw = batch // NW
    xw = jnp.swapaxes(x.astype(jnp.int32), 0, 1)
    xw = xw.reshape(hist, NW, bpw).transpose(1, 0, 2)
    vocab = embedding.shape[0]
    w = jnp.arange(NW)[:, None, None]
    h = jnp.arange(hist)[None, :, None]
    b = jnp.arange(bpw)[None, None, :]
    chunk = h * (bpw // CB) + b // CB
    xw = (xw + ((w + chunk) % KREP) * vocab).reshape(-1)
    table_rep = jnp.tile(embedding, (KREP, 1))

    mesh = plsc.VectorSubcoreMesh(
        core_axis_name="c", subcore_axis_name="s",
        num_cores=NC, num_subcores=NS)
    import functools
    gather = pl.kernel(
        functools.partial(_body, hist, bpw),
        out_type=jax.ShapeDtypeStruct((hist, batch, EMBED), jnp.float32),
        mesh=mesh,
        scratch_types=[
            pltpu.VMEM((hist * bpw,), jnp.int32),
            pltpu.VMEM((2, CB, EMBED), jnp.float32),
            pltpu.SemaphoreType.DMA,
            pltpu.SemaphoreType.DMA,
            pltpu.SemaphoreType.DMA,
            pltpu.SemaphoreType.DMA,
        ],
    )
    out = gather(table_rep, xw)
    return jnp.transpose(out, (1, 0, 2))
